# Initial kernel scaffold; baseline (speedup 1.0000x reference)
import jax
import jax.numpy as jnp
from jax.experimental import pallas as pl


def kernel(x, edge_index, W1, b1, g1, bt1, W2, b2, g2, bt2, W3, b3):
    raise NotImplementedError("write your pallas kernel here")


# speedup vs baseline: 11.3531x; 11.3531x over previous
import functools

import jax
import jax.numpy as jnp
from jax import lax
from jax.experimental import pallas as pl
from jax.experimental.pallas import tpu as pltpu, tpu_sc as plsc

N = 10000
E = 320000
D = 128
NW = 32
EPW = E // NW
CHUNK = 80
NCHUNK = EPW // CHUNK
NPAD = 10240
RPT = NPAD // 16
OCH = 128


def _sc_mesh():
    return plsc.VectorSubcoreMesh(core_axis_name="c", subcore_axis_name="s")


def _sc_degree(dst):
    @functools.partial(
        pl.kernel,
        out_type=jax.ShapeDtypeStruct((NW * N,), jnp.float32),
        mesh=_sc_mesh(),
        compiler_params=pltpu.CompilerParams(needs_layout_passes=False),
        scratch_types=[
            pltpu.VMEM((EPW,), jnp.int32),
            pltpu.VMEM((N,), jnp.float32),
        ],
    )
    def deg_kernel(dst_hbm, out_hbm, dbuf, degt):
        cid = lax.axis_index("c")
        sid = lax.axis_index("s")
        wid = cid * 16 + sid
        z16 = jnp.zeros((16,), jnp.float32)

        def zstep(i, carry):
            degt[pl.ds(i * 16, 16)] = z16
            return carry

        lax.fori_loop(0, N // 16, zstep, 0)
        pltpu.sync_copy(dst_hbm.at[pl.ds(wid * EPW, EPW)], dbuf)
        ones = jnp.ones((16,), jnp.float32)

        def step(i, carry):
            idx = dbuf[pl.ds(i * 16, 16)]
            plsc.addupdate_scatter(degt, [idx], ones)
            return carry

        lax.fori_loop(0, EPW // 16, step, 0)
        pltpu.sync_copy(degt, out_hbm.at[pl.ds(wid * N, N)])

    return deg_kernel(dst)


def _sc_spmm(y, src, dst, rowidx):
    @functools.partial(
        pl.kernel,
        out_type=jax.ShapeDtypeStruct((2 * NPAD, D), jnp.float32),
        mesh=_sc_mesh(),
        compiler_params=pltpu.CompilerParams(needs_layout_passes=False),
        scratch_types=[
            pltpu.VMEM((CHUNK,), jnp.int32),
            pltpu.VMEM((CHUNK,), jnp.int32),
            pltpu.VMEM((CHUNK, D), jnp.float32),
            pltpu.VMEM((OCH, D), jnp.float32),
            pltpu.VMEM((OCH,), jnp.int32),
            pltpu.VMEM_SHARED((NPAD, D), jnp.float32),
            pltpu.SemaphoreType.DMA,
        ],
    )
    def spmm_kernel(y_hbm, src_hbm, dst_hbm, ridx_hbm, out_hbm,
                    idx_s, idx_d, rows, slab, oidx, accum, sem):
        cid = lax.axis_index("c")
        sid = lax.axis_index("s")

        z16 = jnp.zeros((16,), jnp.float32)

        def zstep(i, carry):
            slab[i // 8, pl.ds((i % 8) * 16, 16)] = z16
            return carry

        lax.fori_loop(0, OCH * D // 16, zstep, 0)
        for j in range(RPT // OCH):
            pltpu.sync_copy(slab, accum.at[pl.ds(sid * RPT + j * OCH, OCH)])
        plsc.subcore_barrier()

        ebase = (cid * 16 + sid) * EPW

        def step(i, carry):
            b = ebase + i * CHUNK
            pltpu.sync_copy(src_hbm.at[pl.ds(b, CHUNK)], idx_s)
            pltpu.sync_copy(dst_hbm.at[pl.ds(b, CHUNK)], idx_d)
            pltpu.async_copy(y_hbm.at[idx_s], rows, sem).wait()
            pltpu.sync_copy(rows, accum.at[idx_d], add=True)
            return carry

        lax.fori_loop(0, NCHUNK, step, 0)
        plsc.subcore_barrier()

        for j in range(RPT // OCH):
            start = sid * RPT + j * OCH
            pltpu.sync_copy(accum.at[pl.ds(start, OCH)], slab)
            pltpu.sync_copy(ridx_hbm.at[pl.ds(cid * NPAD + start, OCH)], oidx)
            pltpu.sync_copy(slab, out_hbm.at[oidx])

    return spmm_kernel(y, src, dst, rowidx)


def _tc_mm(x, w):
    def body(x_ref, w_ref, o_ref):
        o_ref[...] = jnp.dot(x_ref[...], w_ref[...],
                             preferred_element_type=jnp.float32,
                             precision=lax.Precision.HIGHEST)

    return pl.pallas_call(
        body,
        out_shape=jax.ShapeDtypeStruct((x.shape[0], w.shape[1]), jnp.float32),
    )(x, w)


def _tc_prep(degp, xw):
    def body(degp_ref, xw_ref, dis_ref, y_ref):
        deg = jnp.sum(degp_ref[...], axis=1, keepdims=True) + 1.0
        dis = lax.rsqrt(deg)
        dis_ref[...] = dis
        y_ref[...] = xw_ref[...] * dis

    return pl.pallas_call(
        body,
        out_shape=[
            jax.ShapeDtypeStruct((N, 1), jnp.float32),
            jax.ShapeDtypeStruct(xw.shape, jnp.float32),
        ],
    )(degp, xw)


def _tc_stage(S0, S1, y, dis, b, g, bt, w_next):
    def body(s0_ref, s1_ref, y_ref, dis_ref, b_ref, g_ref, bt_ref, w_ref,
             yn_ref):
        dis = dis_ref[...]
        c = dis * (s0_ref[...] + s1_ref[...] + y_ref[...]) + b_ref[...]
        mu = jnp.mean(c, axis=0, keepdims=True)
        var = jnp.mean((c - mu) ** 2, axis=0, keepdims=True)
        h = (c - mu) * lax.rsqrt(var + 1e-5) * g_ref[...] + bt_ref[...]
        h = jnp.maximum(h, 0.0)
        yn_ref[...] = dis * jnp.dot(h, w_ref[...],
                                    preferred_element_type=jnp.float32,
                                    precision=lax.Precision.HIGHEST)

    return pl.pallas_call(
        body,
        out_shape=jax.ShapeDtypeStruct((N, w_next.shape[1]), jnp.float32),
    )(S0, S1, y, dis, b, g, bt, w_next)


def _tc_stage_nomm(S0, S1, y, dis, b, g, bt):
    def body(s0_ref, s1_ref, y_ref, dis_ref, b_ref, g_ref, bt_ref, yn_ref):
        dis = dis_ref[...]
        c = dis * (s0_ref[...] + s1_ref[...] + y_ref[...]) + b_ref[...]
        mu = jnp.mean(c, axis=0, keepdims=True)
        var = jnp.mean((c - mu) ** 2, axis=0, keepdims=True)
        h = (c - mu) * lax.rsqrt(var + 1e-5) * g_ref[...] + bt_ref[...]
        yn_ref[...] = dis * jnp.maximum(h, 0.0)

    return pl.pallas_call(
        body,
        out_shape=jax.ShapeDtypeStruct((N, D), jnp.float32),
    )(S0, S1, y, dis, b, g, bt)


def _tc_final(S0, S1, y, dis, w, b):
    def body(s0_ref, s1_ref, y_ref, dis_ref, w_ref, b_ref, o_ref):
        t = s0_ref[...] + s1_ref[...] + y_ref[...]
        logits = dis_ref[...] * jnp.dot(t, w_ref[...],
                                        preferred_element_type=jnp.float32,
                                        precision=lax.Precision.HIGHEST)
        logits = logits + b_ref[...]
        m = jnp.max(logits, axis=1, keepdims=True)
        lse = m + jnp.log(jnp.sum(jnp.exp(logits - m), axis=1, keepdims=True))
        o_ref[...] = logits - lse

    return pl.pallas_call(
        body,
        out_shape=jax.ShapeDtypeStruct((N, w.shape[1]), jnp.float32),
    )(S0, S1, y, dis, w, b)


def kernel(x, edge_index, W1, b1, g1, bt1, W2, b2, g2, bt2, W3, b3):
    src = edge_index[0].astype(jnp.int32)
    dst = edge_index[1].astype(jnp.int32)
    rowidx = jnp.arange(2 * NPAD, dtype=jnp.int32)

    degp = _sc_degree(dst).reshape(NW, N).T
    xw1 = _tc_mm(x, W1)
    dis, y1 = _tc_prep(degp, xw1)

    P = _sc_spmm(y1, src, dst, rowidx)
    y2 = _tc_stage(P[:N], P[NPAD:NPAD + N], y1, dis, b1.reshape(1, -1),
                   g1.reshape(1, -1), bt1.reshape(1, -1), W2)

    P = _sc_spmm(y2, src, dst, rowidx)
    y3 = _tc_stage_nomm(P[:N], P[NPAD:NPAD + N], y2, dis, b2.reshape(1, -1),
                        g2.reshape(1, -1), bt2.reshape(1, -1))

    P = _sc_spmm(y3, src, dst, rowidx)
    return _tc_final(P[:N], P[NPAD:NPAD + N], y3, dis, W3,
                     b3.reshape(1, -1))
